# baseline (speedup 1.0000x reference)
import functools

import jax
import jax.numpy as jnp
from jax import lax
from jax.experimental import pallas as pl
from jax.experimental.pallas import tpu as pltpu
from jax.experimental.pallas import tpu_sc as plsc

NEG_SLOPE = 0.01
NC = 2
NS = 16
NW = NC * NS
CHUNK = 128
LANES = 16


def _leaky(v):
    return jnp.where(v >= 0, v, NEG_SLOPE * v)


def _matmul_body(x_ref, wt_ref, h_ref):
    h = jnp.dot(x_ref[...], wt_ref[...], preferred_element_type=jnp.float32)
    h_ref[...] = _leaky(h)


def _combine_body(p_ref, o_ref):
    o_ref[...] = _leaky(p_ref[0] + p_ref[1])


def _make_aggregate(n_nodes, d, nwin):
    stripe = (n_nodes // NS) // 8 * 8
    tail = n_nodes - stripe * NS
    zcopies = stripe // CHUNK
    zrem = stripe - zcopies * CHUNK
    dp = d // 2
    assert nwin % 4 == 0 and nwin >= 8

    mesh = plsc.VectorSubcoreMesh(
        core_axis_name="c", subcore_axis_name="s", num_cores=NC, num_subcores=NS
    )

    @functools.partial(
        pl.kernel,
        out_type=jax.ShapeDtypeStruct((NC, n_nodes, d), jnp.float32),
        mesh=mesh,
        compiler_params=pltpu.CompilerParams(needs_layout_passes=False,
                                             use_tc_tiling_on_sc=False),
        scratch_types=[
            [pltpu.VMEM((CHUNK, dp), jnp.int32) for _ in range(2)],
            [pltpu.VMEM((CHUNK, d), jnp.float32) for _ in range(2)],
            [pltpu.VMEM((CHUNK,), jnp.int32) for _ in range(2)],
            [pltpu.VMEM((CHUNK,), jnp.int32) for _ in range(4)],
            [pltpu.VMEM((CHUNK,), jnp.float32) for _ in range(4)],
            pltpu.VMEM_SHARED((n_nodes, d), jnp.float32),
            [pltpu.SemaphoreType.DMA for _ in range(14)],
        ],
    )
    def aggregate(hp_hbm, row_hbm, col_hbm, w_hbm, out_hbm,
                  gath_s, stag_s, col_s, row_s, w_s, acc, sems):
        c = lax.axis_index("c")
        s = lax.axis_index("s")
        wid = s * NC + c
        gsem = sems[0:2]
        ssem = sems[2:4]
        csem = sems[4:6]
        rsem = sems[6:10]
        wsem = sems[10:14]
        ebase = wid * (nwin * CHUNK)

        def zero_body(e, _):
            for dd in range(d // LANES):
                stag_s[0][e, pl.ds(dd * LANES, LANES)] = jnp.zeros(
                    (LANES,), jnp.float32)
            return 0
        lax.fori_loop(jnp.int32(0), jnp.int32(CHUNK), zero_body, 0,
                      unroll=False)

        zbase = pl.multiple_of(s * stripe, 8)
        for k in range(zcopies):
            pltpu.sync_copy(stag_s[0], acc.at[pl.ds(zbase + k * CHUNK, CHUNK)])
        if zrem:
            pltpu.sync_copy(stag_s[0].at[pl.ds(0, zrem)],
                            acc.at[pl.ds(zbase + zcopies * CHUNK, zrem)])
        if tail:
            @pl.when(s == NS - 1)
            def _zero_tail():
                pltpu.sync_copy(stag_s[0].at[pl.ds(0, tail)],
                                acc.at[pl.ds(stripe * NS, tail)])
        plsc.subcore_barrier()

        def start_col(j, b):
            pltpu.async_copy(col_hbm.at[pl.ds(ebase + j * CHUNK, CHUNK)],
                             col_s[b], csem[b])

        def wait_col(j, b):
            pltpu.make_async_copy(
                col_hbm.at[pl.ds(ebase + j * CHUNK, CHUNK)],
                col_s[b], csem[b]).wait()

        def start_rw(j, b):
            pltpu.async_copy(row_hbm.at[pl.ds(ebase + j * CHUNK, CHUNK)],
                             row_s[b], rsem[b])
            pltpu.async_copy(w_hbm.at[pl.ds(ebase + j * CHUNK, CHUNK)],
                             w_s[b], wsem[b])

        def wait_rw(j, b):
            pltpu.make_async_copy(
                row_hbm.at[pl.ds(ebase + j * CHUNK, CHUNK)],
                row_s[b], rsem[b]).wait()
            pltpu.make_async_copy(
                w_hbm.at[pl.ds(ebase + j * CHUNK, CHUNK)],
                w_s[b], wsem[b]).wait()

        def start_gather(b):
            pltpu.async_copy(hp_hbm.at[col_s[b]], gath_s[b], gsem[b])

        def wait_gather(b):
            pltpu.make_async_copy(hp_hbm.at[col_s[b]], gath_s[b],
                                  gsem[b]).wait()

        def start_scatter(b, rb):
            pltpu.async_copy(stag_s[b], acc.at[row_s[rb]], ssem[b], add=True)

        def wait_scatter(b, rb):
            pltpu.make_async_copy(stag_s[b], acc.at[row_s[rb]],
                                  ssem[b]).wait()

        def scale(b, rb):
            gbuf = gath_s[b]
            sbuf = stag_s[b]

            def group(g, _):
                wvec = w_s[rb][pl.ds(g * LANES, LANES)]
                for l in range(LANES):
                    e = g * LANES + l
                    wv = wvec[l]
                    for gg in range(d // 32):
                        v = gbuf[e, pl.ds(gg * LANES, LANES)]
                        lo, hi = plsc.unpack(
                            plsc.bitcast(v, jnp.bfloat16),
                            format=plsc.PackFormat.INTERLEAVED)
                        sbuf[e, pl.ds(gg * 32, LANES)] = lo * wv
                        sbuf[e, pl.ds(gg * 32 + LANES, LANES)] = hi * wv
                return 0
            lax.fori_loop(jnp.int32(0), jnp.int32(CHUNK // LANES), group, 0,
                          unroll=False)

        def window(j, q, r4, first, do_pref, do_next):
            if not first:
                wait_scatter(q, (r4 + 2) % 4)
            if do_pref:
                start_rw(j + 2, (r4 + 2) % 4)
            wait_gather(q)
            if do_pref:
                start_col(j + 2, q)
            if do_next:
                wait_col(j + 1, 1 - q)
                start_gather(1 - q)
            wait_rw(j, r4)
            scale(q, r4)
            start_scatter(q, r4)

        z = jnp.int32(0)
        start_col(z, 0)
        start_col(z + 1, 1)
        start_rw(z, 0)
        start_rw(z + 1, 1)
        wait_col(z, 0)
        start_gather(0)

        window(z, 0, 0, True, True, True)
        window(z + 1, 1, 1, True, True, True)

        def block(i, _):
            jb = i * 4 + 2
            for p in range(4):
                window(jb + p, p % 2, (2 + p) % 4, False, True, True)
            return 0
        lax.fori_loop(jnp.int32(0), jnp.int32((nwin - 4) // 4), block, 0,
                      unroll=False)

        jt = jnp.int32(nwin - 2)
        window(jt, 0, 2, False, False, True)
        window(jt + 1, 1, 3, False, False, False)

        wait_scatter(0, 2)
        wait_scatter(1, 3)
        plsc.subcore_barrier()

        wbase = pl.multiple_of(s * stripe, 8)
        pltpu.sync_copy(acc.at[pl.ds(wbase, stripe)],
                        out_hbm.at[c, pl.ds(wbase, stripe)])
        if tail:
            @pl.when(s == NS - 1)
            def _write_tail():
                pltpu.sync_copy(acc.at[pl.ds(stripe * NS, tail)],
                                out_hbm.at[c, pl.ds(stripe * NS, tail)])

    return aggregate


def kernel(input, edge_index, edge_weight, W):
    n, d_in = input.shape
    d_out = W.shape[0]
    e = edge_index.shape[1]

    row = edge_index[0].astype(jnp.int32)
    col = edge_index[1].astype(jnp.int32)
    w = edge_weight.astype(jnp.float32)

    tile_quantum = NW * CHUNK * 4
    e_pad = ((e + tile_quantum - 1) // tile_quantum) * tile_quantum
    pad = e_pad - e
    if pad:
        pad_idx = jnp.arange(pad, dtype=jnp.int32) % n
        row = jnp.concatenate([row, pad_idx])
        col = jnp.concatenate([col, pad_idx])
        w = jnp.concatenate([w, jnp.zeros((pad,), jnp.float32)])
    nwin = e_pad // (NW * CHUNK)

    h = pl.pallas_call(
        _matmul_body,
        out_shape=jax.ShapeDtypeStruct((n, d_out), jnp.float32),
    )(input, W.T)

    hb = h.astype(jnp.bfloat16).reshape(n, d_out // 32, 2, 16)
    pairs = jnp.stack([hb[:, :, 0, :], hb[:, :, 1, :]], axis=-1)
    h_pack = jax.lax.bitcast_convert_type(pairs, jnp.int32).reshape(
        n, d_out // 2)

    partials = _make_aggregate(n, d_out, nwin)(h_pack, row, col, w)

    out = pl.pallas_call(
        _combine_body,
        out_shape=jax.ShapeDtypeStruct((n, d_out), jnp.float32),
    )(partials)
    return out


# speedup vs baseline: 1.7298x; 1.7298x over previous
import functools

import jax
import jax.numpy as jnp
from jax import lax
from jax.experimental import pallas as pl
from jax.experimental.pallas import tpu as pltpu
from jax.experimental.pallas import tpu_sc as plsc

NEG_SLOPE = 0.01
NC = 2
NS = 16
NW = NC * NS
CHUNK = 128
LANES = 16
NSLOT = 3


def _leaky(v):
    return jnp.where(v >= 0, v, NEG_SLOPE * v)


def _matmul_body(x_ref, wt_ref, h_ref):
    h = jnp.dot(x_ref[...], wt_ref[...], preferred_element_type=jnp.float32)
    h_ref[...] = _leaky(h)


def _combine_body(p_ref, o_ref):
    o_ref[...] = _leaky(p_ref[0] + p_ref[1])


def _make_aggregate(n_nodes, d, nwin):
    stripe = (n_nodes // NS) // 8 * 8
    tail = n_nodes - stripe * NS
    zcopies = stripe // CHUNK
    zrem = stripe - zcopies * CHUNK
    assert nwin % NSLOT == 0 and nwin >= 3 * NSLOT

    mesh = plsc.VectorSubcoreMesh(
        core_axis_name="c", subcore_axis_name="s", num_cores=NC, num_subcores=NS
    )

    @functools.partial(
        pl.kernel,
        out_type=jax.ShapeDtypeStruct((NC, n_nodes, d), jnp.float32),
        mesh=mesh,
        scratch_types=[
            [pltpu.VMEM((CHUNK, d), jnp.float32) for _ in range(NSLOT)],
            [pltpu.VMEM((CHUNK,), jnp.int32) for _ in range(NSLOT)],
            [pltpu.VMEM((CHUNK,), jnp.int32) for _ in range(NSLOT)],
            [pltpu.VMEM((CHUNK,), jnp.float32) for _ in range(NSLOT)],
            pltpu.VMEM_SHARED((n_nodes, d), jnp.float32),
            [pltpu.SemaphoreType.DMA for _ in range(5 * NSLOT)],
        ],
    )
    def aggregate(h_hbm, row_hbm, col_hbm, w_hbm, out_hbm,
                  rows_s, col_s, row_s, w_s, acc, sems):
        c = lax.axis_index("c")
        s = lax.axis_index("s")
        wid = s * NC + c
        gsem = sems[0:NSLOT]
        ssem = sems[NSLOT:2 * NSLOT]
        csem = sems[2 * NSLOT:3 * NSLOT]
        rsem = sems[3 * NSLOT:4 * NSLOT]
        wsem = sems[4 * NSLOT:5 * NSLOT]
        ebase = wid * (nwin * CHUNK)

        def zero_body(e, _):
            for dd in range(d // LANES):
                rows_s[0][e, pl.ds(dd * LANES, LANES)] = jnp.zeros(
                    (LANES,), jnp.float32)
            return 0
        lax.fori_loop(jnp.int32(0), jnp.int32(CHUNK), zero_body, 0,
                      unroll=False)

        zbase = pl.multiple_of(s * stripe, 8)
        for k in range(zcopies):
            pltpu.sync_copy(rows_s[0], acc.at[pl.ds(zbase + k * CHUNK, CHUNK)])
        if zrem:
            pltpu.sync_copy(rows_s[0].at[pl.ds(0, zrem)],
                            acc.at[pl.ds(zbase + zcopies * CHUNK, zrem)])
        if tail:
            @pl.when(s == NS - 1)
            def _zero_tail():
                pltpu.sync_copy(rows_s[0].at[pl.ds(0, tail)],
                                acc.at[pl.ds(stripe * NS, tail)])
        plsc.subcore_barrier()

        def start_col(j, b):
            pltpu.async_copy(col_hbm.at[pl.ds(ebase + j * CHUNK, CHUNK)],
                             col_s[b], csem[b])

        def wait_col(j, b):
            pltpu.make_async_copy(
                col_hbm.at[pl.ds(ebase + j * CHUNK, CHUNK)],
                col_s[b], csem[b]).wait()

        def start_rw(j, b):
            pltpu.async_copy(row_hbm.at[pl.ds(ebase + j * CHUNK, CHUNK)],
                             row_s[b], rsem[b])
            pltpu.async_copy(w_hbm.at[pl.ds(ebase + j * CHUNK, CHUNK)],
                             w_s[b], wsem[b])

        def wait_rw(j, b):
            pltpu.make_async_copy(
                row_hbm.at[pl.ds(ebase + j * CHUNK, CHUNK)],
                row_s[b], rsem[b]).wait()
            pltpu.make_async_copy(
                w_hbm.at[pl.ds(ebase + j * CHUNK, CHUNK)],
                w_s[b], wsem[b]).wait()

        def start_gather(b):
            pltpu.async_copy(h_hbm.at[col_s[b]], rows_s[b], gsem[b])

        def wait_gather(b):
            pltpu.make_async_copy(h_hbm.at[col_s[b]], rows_s[b],
                                  gsem[b]).wait()

        def start_scatter(b):
            pltpu.async_copy(rows_s[b], acc.at[row_s[b]], ssem[b], add=True)

        def wait_scatter(b):
            pltpu.make_async_copy(rows_s[b], acc.at[row_s[b]], ssem[b]).wait()

        def scale(b):
            rbuf = rows_s[b]

            def group(g, _):
                wvec = w_s[b][pl.ds(g * LANES, LANES)]
                for l in range(LANES):
                    e = g * LANES + l
                    wv = wvec[l]
                    for dd in range(d // LANES):
                        sl = pl.ds(dd * LANES, LANES)
                        rbuf[e, sl] = rbuf[e, sl] * wv
                return 0
            lax.fori_loop(jnp.int32(0), jnp.int32(CHUNK // LANES), group, 0,
                          unroll=False)

        def window(j, b, wait_prev, pref_rw, pref_col, nxt):
            bn = (b + 1) % NSLOT
            bn2 = (b + 2) % NSLOT
            if wait_prev:
                wait_scatter(bn)
            if pref_rw:
                start_rw(j + 1, bn)
            if pref_col:
                start_col(j + 2, bn2)
            wait_gather(b)
            if nxt:
                wait_col(j + 1, bn)
                start_gather(bn)
            wait_rw(j, b)
            scale(b)
            start_scatter(b)

        z = jnp.int32(0)
        start_col(z, 0)
        start_col(z + 1, 1)
        start_rw(z, 0)
        start_rw(z + 1, 1)
        wait_col(z, 0)
        start_gather(0)

        window(z, 0, False, False, True, True)
        window(z + 1, 1, False, True, True, True)
        window(z + 2, 2, True, True, True, True)

        def block(i, _):
            jb = i * NSLOT + 3
            for p in range(NSLOT):
                window(jb + p, p, True, True, True, True)
            return 0
        lax.fori_loop(jnp.int32(0), jnp.int32((nwin - 6) // NSLOT), block, 0,
                      unroll=False)

        jt = jnp.int32(nwin - 3)
        window(jt, 0, True, True, True, True)
        window(jt + 1, 1, True, True, False, True)
        window(jt + 2, 2, True, False, False, False)

        wait_scatter(1)
        wait_scatter(2)
        plsc.subcore_barrier()

        wbase = pl.multiple_of(s * stripe, 8)
        pltpu.sync_copy(acc.at[pl.ds(wbase, stripe)],
                        out_hbm.at[c, pl.ds(wbase, stripe)])
        if tail:
            @pl.when(s == NS - 1)
            def _write_tail():
                pltpu.sync_copy(acc.at[pl.ds(stripe * NS, tail)],
                                out_hbm.at[c, pl.ds(stripe * NS, tail)])

    return aggregate


def kernel(input, edge_index, edge_weight, W):
    n, d_in = input.shape
    d_out = W.shape[0]
    e = edge_index.shape[1]

    row = edge_index[0].astype(jnp.int32)
    col = edge_index[1].astype(jnp.int32)
    w = edge_weight.astype(jnp.float32)

    tile_quantum = NW * CHUNK * NSLOT
    e_pad = ((e + tile_quantum - 1) // tile_quantum) * tile_quantum
    e_pad = max(e_pad, NW * CHUNK * 3 * NSLOT)
    pad = e_pad - e
    if pad:
        pad_idx = jnp.arange(pad, dtype=jnp.int32) % n
        row = jnp.concatenate([row, pad_idx])
        col = jnp.concatenate([col, pad_idx])
        w = jnp.concatenate([w, jnp.zeros((pad,), jnp.float32)])
    nwin = e_pad // (NW * CHUNK)

    h = pl.pallas_call(
        _matmul_body,
        out_shape=jax.ShapeDtypeStruct((n, d_out), jnp.float32),
    )(input, W.T)

    partials = _make_aggregate(n, d_out, nwin)(h, row, col, w)

    out = pl.pallas_call(
        _combine_body,
        out_shape=jax.ShapeDtypeStruct((n, d_out), jnp.float32),
    )(partials)
    return out


# speedup vs baseline: 1.7472x; 1.0101x over previous
import functools

import jax
import jax.numpy as jnp
from jax import lax
from jax.experimental import pallas as pl
from jax.experimental.pallas import tpu as pltpu
from jax.experimental.pallas import tpu_sc as plsc

NEG_SLOPE = 0.01
NC = 2
NS = 16
NW = NC * NS
CHUNK = 128
LANES = 16
NSLOT = 3


def _leaky(v):
    return jnp.where(v >= 0, v, NEG_SLOPE * v)


def _matmul_body(x_ref, wt_ref, h_ref):
    h = jnp.dot(x_ref[...], wt_ref[...], preferred_element_type=jnp.float32)
    h_ref[...] = _leaky(h)


def _combine_body(p_ref, o_ref):
    o_ref[...] = _leaky(p_ref[0] + p_ref[1])


def _make_aggregate(n_nodes, d, nwin):
    stripe = (n_nodes // NS) // 8 * 8
    tail = n_nodes - stripe * NS
    zcopies = stripe // CHUNK
    zrem = stripe - zcopies * CHUNK
    assert nwin % NSLOT == 0 and nwin // NSLOT >= 2

    mesh = plsc.VectorSubcoreMesh(
        core_axis_name="c", subcore_axis_name="s", num_cores=NC, num_subcores=NS
    )

    @functools.partial(
        pl.kernel,
        out_type=jax.ShapeDtypeStruct((NC, n_nodes, d), jnp.float32),
        mesh=mesh,
        scratch_types=[
            [pltpu.VMEM((CHUNK, d), jnp.float32) for _ in range(NSLOT)],
            [pltpu.VMEM((CHUNK,), jnp.int32) for _ in range(NSLOT)],
            [pltpu.VMEM((CHUNK,), jnp.int32) for _ in range(NSLOT)],
            [pltpu.VMEM((CHUNK,), jnp.float32) for _ in range(NSLOT)],
            pltpu.VMEM_SHARED((n_nodes, d), jnp.float32),
            [pltpu.SemaphoreType.DMA for _ in range(5 * NSLOT)],
        ],
    )
    def aggregate(h_hbm, row_hbm, col_hbm, w_hbm, out_hbm,
                  rows_s, col_s, row_s, w_s, acc, sems):
        c = lax.axis_index("c")
        s = lax.axis_index("s")
        wid = s * NC + c
        gsem = sems[0:NSLOT]
        ssem = sems[NSLOT:2 * NSLOT]
        csem = sems[2 * NSLOT:3 * NSLOT]
        rsem = sems[3 * NSLOT:4 * NSLOT]
        wsem = sems[4 * NSLOT:5 * NSLOT]
        ebase = wid * (nwin * CHUNK)

        def zero_body(e, _):
            for dd in range(d // LANES):
                rows_s[0][e, pl.ds(dd * LANES, LANES)] = jnp.zeros(
                    (LANES,), jnp.float32)
            return 0
        lax.fori_loop(jnp.int32(0), jnp.int32(CHUNK), zero_body, 0,
                      unroll=False)

        zbase = pl.multiple_of(s * stripe, 8)
        for k in range(zcopies):
            pltpu.sync_copy(rows_s[0], acc.at[pl.ds(zbase + k * CHUNK, CHUNK)])
        if zrem:
            pltpu.sync_copy(rows_s[0].at[pl.ds(0, zrem)],
                            acc.at[pl.ds(zbase + zcopies * CHUNK, zrem)])
        if tail:
            @pl.when(s == NS - 1)
            def _zero_tail():
                pltpu.sync_copy(rows_s[0].at[pl.ds(0, tail)],
                                acc.at[pl.ds(stripe * NS, tail)])
        plsc.subcore_barrier()

        def start_col(j, b):
            pltpu.async_copy(col_hbm.at[pl.ds(ebase + j * CHUNK, CHUNK)],
                             col_s[b], csem[b])

        def wait_col(j, b):
            pltpu.make_async_copy(
                col_hbm.at[pl.ds(ebase + j * CHUNK, CHUNK)],
                col_s[b], csem[b]).wait()

        def start_rw(j, b):
            pltpu.async_copy(row_hbm.at[pl.ds(ebase + j * CHUNK, CHUNK)],
                             row_s[b], rsem[b])
            pltpu.async_copy(w_hbm.at[pl.ds(ebase + j * CHUNK, CHUNK)],
                             w_s[b], wsem[b])

        def wait_rw(j, b):
            pltpu.make_async_copy(
                row_hbm.at[pl.ds(ebase + j * CHUNK, CHUNK)],
                row_s[b], rsem[b]).wait()
            pltpu.make_async_copy(
                w_hbm.at[pl.ds(ebase + j * CHUNK, CHUNK)],
                w_s[b], wsem[b]).wait()

        def start_gather(b):
            pltpu.async_copy(h_hbm.at[col_s[b]], rows_s[b], gsem[b])

        def wait_gather(b):
            pltpu.make_async_copy(h_hbm.at[col_s[b]], rows_s[b],
                                  gsem[b]).wait()

        def start_scatter(b):
            pltpu.async_copy(rows_s[b], acc.at[row_s[b]], ssem[b], add=True)

        def wait_scatter(b):
            pltpu.make_async_copy(rows_s[b], acc.at[row_s[b]], ssem[b]).wait()

        def scale(b):
            rbuf = rows_s[b]

            def group(g, _):
                wvec = w_s[b][pl.ds(g * LANES, LANES)]
                for l in range(LANES):
                    e = g * LANES + l
                    wv = wvec[l]
                    for dd in range(d // LANES):
                        sl = pl.ds(dd * LANES, LANES)
                        rbuf[e, sl] = rbuf[e, sl] * wv
                return 0
            lax.fori_loop(jnp.int32(0), jnp.int32(CHUNK // LANES), group, 0,
                          unroll=False)

        z = jnp.int32(0)
        start_col(z, 0)
        start_col(z + 1, 1)
        start_rw(z, 0)
        start_rw(z + 1, 1)
        wait_col(z, 0)
        start_gather(0)

        def block(i, _):
            for b in range(NSLOT):
                j = i * NSLOT + b
                bn = (b + 1) % NSLOT
                bn2 = (b + 2) % NSLOT

                @pl.when(j >= 2)
                def _free_next_slot():
                    wait_scatter(bn)

                @pl.when(jnp.logical_and(j >= 1, j + 1 < nwin))
                def _prefetch_rw():
                    start_rw(j + 1, bn)

                @pl.when(j + 2 < nwin)
                def _prefetch_col():
                    start_col(j + 2, bn2)

                wait_gather(b)

                @pl.when(j + 1 < nwin)
                def _next_gather():
                    wait_col(j + 1, bn)
                    start_gather(bn)

                wait_rw(j, b)
                scale(b)
                start_scatter(b)
            return 0
        lax.fori_loop(jnp.int32(0), jnp.int32(nwin // NSLOT), block, 0,
                      unroll=False)

        wait_scatter((nwin - 2) % NSLOT)
        wait_scatter((nwin - 1) % NSLOT)
        plsc.subcore_barrier()

        wbase = pl.multiple_of(s * stripe, 8)
        pltpu.sync_copy(acc.at[pl.ds(wbase, stripe)],
                        out_hbm.at[c, pl.ds(wbase, stripe)])
        if tail:
            @pl.when(s == NS - 1)
            def _write_tail():
                pltpu.sync_copy(acc.at[pl.ds(stripe * NS, tail)],
                                out_hbm.at[c, pl.ds(stripe * NS, tail)])

    return aggregate


def kernel(input, edge_index, edge_weight, W):
    n, d_in = input.shape
    d_out = W.shape[0]
    e = edge_index.shape[1]

    row = edge_index[0].astype(jnp.int32)
    col = edge_index[1].astype(jnp.int32)
    w = edge_weight.astype(jnp.float32)

    tile_quantum = NW * CHUNK * NSLOT
    e_pad = ((e + tile_quantum - 1) // tile_quantum) * tile_quantum
    pad = e_pad - e
    if pad:
        pad_idx = jnp.arange(pad, dtype=jnp.int32) % n
        row = jnp.concatenate([row, pad_idx])
        col = jnp.concatenate([col, pad_idx])
        w = jnp.concatenate([w, jnp.zeros((pad,), jnp.float32)])
    nwin = e_pad // (NW * CHUNK)

    h = pl.pallas_call(
        _matmul_body,
        out_shape=jax.ShapeDtypeStruct((n, d_out), jnp.float32),
    )(input, W.T)

    partials = _make_aggregate(n, d_out, nwin)(h, row, col, w)

    out = pl.pallas_call(
        _combine_body,
        out_shape=jax.ShapeDtypeStruct((n, d_out), jnp.float32),
    )(partials)
    return out
